# baseline (speedup 1.0000x reference)
import functools
import math

import jax
import jax.numpy as jnp
from jax.experimental import pallas as pl
from jax.experimental.pallas import tpu as pltpu
from jax.experimental.pallas import tpu_sc as plsc

_D_MODEL = 768
_N_EXP = 64
_Z_COEF = 0.001
_AUX_COEF = 0.01
_CAP_FACTOR = 1.0
_MIN_CAP = 4

_TB = 4096
_LANES = 128
_SC_TILES = 16


def _router_body(x_ref, wt_ref, idx_ref, prob_ref, stats_ref,
                 p_acc, z_acc):
    i = pl.program_id(0)
    nb = pl.num_programs(0)
    tb = x_ref.shape[0]

    lg = jax.lax.dot_general(
        wt_ref[...], x_ref[...],
        dimension_numbers=(((0,), (1,)), ((), ())),
        preferred_element_type=jnp.float32)
    m = jnp.max(lg, axis=0, keepdims=True)
    e = jnp.exp(lg - m)
    s = jnp.sum(e, axis=0, keepdims=True)
    eid = jax.lax.broadcasted_iota(jnp.int32, lg.shape, 0)
    amax = jnp.min(jnp.where(lg >= m, eid, _N_EXP), axis=0)
    idx_ref[...] = amax
    prob_ref[...] = 1.0 / s[0]
    lse = m[0] + jnp.log(s[0])

    p_blk = jnp.sum((e * (1.0 / s)).reshape(_N_EXP, tb // _LANES, _LANES),
                    axis=1)
    z_blk = jnp.sum((lse * lse).reshape(tb // _LANES, _LANES),
                    axis=0, keepdims=True)

    @pl.when(i == 0)
    def _init():
        p_acc[...] = jnp.zeros_like(p_acc)
        z_acc[...] = jnp.zeros_like(z_acc)

    p_acc[...] += p_blk
    z_acc[...] += z_blk

    @pl.when(i == nb - 1)
    def _finish():
        p_vec = jnp.sum(p_acc[...], axis=1)
        z_sum = jnp.sum(z_acc[...])
        stats_ref[...] = jnp.concatenate(
            [p_vec, jnp.full((_N_EXP,), z_sum, jnp.float32)])


def _sc_body(idx_hbm, stats_hbm, cnt_hbm, aux_hbm,
             idx_v, ones_v, stats_v, cfin, auxst, shared, sem,
             *, rows_per_tile, n_tokens):
    cid = jax.lax.axis_index("c")
    sid = jax.lax.axis_index("s")

    @pl.when(cid == 0)
    def _load():
        pltpu.sync_copy(idx_hbm.at[pl.ds(sid * rows_per_tile, rows_per_tile)],
                        idx_v)
        for t in range(_LANES // 16):
            ones_v[pl.ds(t * 16, 16)] = jnp.ones((16,), jnp.int32)

    @pl.when((cid == 0) & (sid == 0))
    def _zero():
        for k in range(_N_EXP // 16):
            cfin[pl.ds(k * 16, 16)] = jnp.zeros((16,), jnp.int32)
        pltpu.sync_copy(cfin, shared)
        pltpu.sync_copy(stats_hbm, stats_v)

    plsc.subcore_barrier()

    @pl.when(cid == 0)
    def _scatter():
        copies = [
            pltpu.make_async_copy(ones_v, shared.at[idx_v.at[j]], sem)
            for j in range(rows_per_tile)
        ]
        for c in copies:
            c.start(add=True)
        for c in copies:
            c.wait()

    plsc.subcore_barrier()

    @pl.when((cid == 0) & (sid == 0))
    def _final():
        pltpu.sync_copy(shared, cfin)
        acc = jnp.zeros((16,), jnp.float32)
        for k in range(_N_EXP // 16):
            acc = acc + (cfin[pl.ds(k * 16, 16)].astype(jnp.float32)
                         * stats_v[pl.ds(k * 16, 16)])
        fp_dot = jnp.sum(acc)
        zv = stats_v[pl.ds(_N_EXP, 16)]
        inv_n = 1.0 / n_tokens
        auxst[...] = (jnp.full((16,), _AUX_COEF * _N_EXP * inv_n * inv_n
                               * fp_dot, jnp.float32)
                      + zv * (_Z_COEF * inv_n))
        pltpu.sync_copy(cfin, cnt_hbm)
        pltpu.sync_copy(auxst, aux_hbm)


def kernel(x, W):
    B, T, D = x.shape
    n = B * T
    x_flat = x.reshape(n, D)
    wt = W.T
    nb = n // _TB

    idx, prob, stats = pl.pallas_call(
        _router_body,
        grid=(nb,),
        in_specs=[
            pl.BlockSpec((_TB, D), lambda i: (i, 0)),
            pl.BlockSpec((D, _N_EXP), lambda i: (0, 0)),
        ],
        out_specs=[
            pl.BlockSpec((_TB,), lambda i: (i,)),
            pl.BlockSpec((_TB,), lambda i: (i,)),
            pl.BlockSpec((2 * _N_EXP,), lambda i: (0,)),
        ],
        out_shape=[
            jax.ShapeDtypeStruct((n,), jnp.int32),
            jax.ShapeDtypeStruct((n,), jnp.float32),
            jax.ShapeDtypeStruct((2 * _N_EXP,), jnp.float32),
        ],
        scratch_shapes=[
            pltpu.VMEM((_N_EXP, _LANES), jnp.float32),
            pltpu.VMEM((1, _LANES), jnp.float32),
        ],
    )(x_flat, wt)

    rows_per_tile = n // (_LANES * _SC_TILES)
    sc_call = pl.kernel(
        functools.partial(_sc_body, rows_per_tile=rows_per_tile,
                          n_tokens=float(n)),
        out_type=[
            jax.ShapeDtypeStruct((_N_EXP,), jnp.int32),
            jax.ShapeDtypeStruct((16,), jnp.float32),
        ],
        mesh=plsc.VectorSubcoreMesh(core_axis_name="c", subcore_axis_name="s"),
        compiler_params=pltpu.CompilerParams(needs_layout_passes=False),
        scratch_types=[
            pltpu.VMEM((rows_per_tile, _LANES), jnp.int32),
            pltpu.VMEM((_LANES,), jnp.int32),
            pltpu.VMEM((2 * _N_EXP,), jnp.float32),
            pltpu.VMEM((_N_EXP,), jnp.int32),
            pltpu.VMEM((16,), jnp.float32),
            pltpu.VMEM_SHARED((_N_EXP,), jnp.int32),
            pltpu.SemaphoreType.DMA,
        ],
    )
    counts, auxv = sc_call(idx.reshape(n // _LANES, _LANES), stats)

    capacity = max(_MIN_CAP, math.ceil(_CAP_FACTOR * n / _N_EXP))
    return (idx, prob, counts, jnp.array(capacity, dtype=jnp.int32),
            auxv[0])


# speedup vs baseline: 1.0075x; 1.0075x over previous
import functools
import math

import jax
import jax.numpy as jnp
from jax.experimental import pallas as pl
from jax.experimental.pallas import tpu as pltpu
from jax.experimental.pallas import tpu_sc as plsc

_D_MODEL = 768
_N_EXP = 64
_Z_COEF = 0.001
_AUX_COEF = 0.01
_CAP_FACTOR = 1.0
_MIN_CAP = 4

_TB = 4096
_LANES = 128
_SC_TILES = 16


def _router_body(x_ref, wt_ref, idx_ref, prob_ref, idx2_ref, stats_ref,
                 p_acc, z_acc, *, tokens_per_tile):
    i = pl.program_id(0)
    nb = pl.num_programs(0)
    tb = x_ref.shape[0]

    lg = jax.lax.dot_general(
        wt_ref[...], x_ref[...],
        dimension_numbers=(((0,), (1,)), ((), ())),
        preferred_element_type=jnp.float32)
    m = jnp.max(lg, axis=0, keepdims=True)
    e = jnp.exp(lg - m)
    s = jnp.sum(e, axis=0, keepdims=True)
    eid = jax.lax.broadcasted_iota(jnp.int32, lg.shape, 0)
    amax = jnp.min(jnp.where(lg >= m, eid, _N_EXP), axis=0)
    idx_ref[...] = amax
    prob_ref[...] = 1.0 / s[0]
    lse = m[0] + jnp.log(s[0])

    gidx = i * tb + jax.lax.broadcasted_iota(jnp.int32, (tb,), 0)
    idx2_ref[...] = amax + (gidx // tokens_per_tile) * _N_EXP

    p_blk = jnp.sum((e * (1.0 / s)).reshape(_N_EXP, tb // _LANES, _LANES),
                    axis=1)
    z_blk = jnp.sum((lse * lse).reshape(tb // _LANES, _LANES),
                    axis=0, keepdims=True)

    @pl.when(i == 0)
    def _init():
        p_acc[...] = jnp.zeros_like(p_acc)
        z_acc[...] = jnp.zeros_like(z_acc)

    p_acc[...] += p_blk
    z_acc[...] += z_blk

    @pl.when(i == nb - 1)
    def _finish():
        p_vec = jnp.sum(p_acc[...], axis=1)
        z_sum = jnp.sum(z_acc[...])
        stats_ref[...] = jnp.concatenate(
            [p_vec, jnp.full((_N_EXP,), z_sum, jnp.float32)])


def _sc_body(idx_hbm, stats_hbm, cnt_hbm, aux_hbm,
             idx_v, ones_v, stats_v, cfin, hall_v, auxst, shared, sem,
             *, rows_per_tile, n_tokens):
    cid = jax.lax.axis_index("c")
    sid = jax.lax.axis_index("s")

    @pl.when(cid == 0)
    def _load():
        pltpu.sync_copy(idx_hbm.at[pl.ds(sid * rows_per_tile, rows_per_tile)],
                        idx_v)
        for t in range(_LANES // 16):
            ones_v[pl.ds(t * 16, 16)] = jnp.ones((16,), jnp.int32)
        for k in range(_N_EXP // 16):
            cfin[pl.ds(k * 16, 16)] = jnp.zeros((16,), jnp.int32)
        pltpu.sync_copy(cfin, shared.at[pl.ds(sid * _N_EXP, _N_EXP)])

    @pl.when((cid == 0) & (sid == 0))
    def _pref():
        pltpu.sync_copy(stats_hbm, stats_v)

    plsc.subcore_barrier()

    @pl.when(cid == 0)
    def _scatter():
        copies = [
            pltpu.make_async_copy(ones_v, shared.at[idx_v.at[j]], sem)
            for j in range(rows_per_tile)
        ]
        for c in copies:
            c.start(add=True)
        for c in copies:
            c.wait()

    plsc.subcore_barrier()

    @pl.when((cid == 0) & (sid == 0))
    def _final():
        pltpu.sync_copy(shared, hall_v)
        tot = []
        for k in range(_N_EXP // 16):
            acc_k = jnp.zeros((16,), jnp.int32)
            for r in range(_SC_TILES):
                acc_k = acc_k + hall_v[pl.ds(r * _N_EXP + k * 16, 16)]
            tot.append(acc_k)
        acc = jnp.zeros((16,), jnp.float32)
        for k in range(_N_EXP // 16):
            cfin[pl.ds(k * 16, 16)] = tot[k]
            acc = acc + (tot[k].astype(jnp.float32)
                         * stats_v[pl.ds(k * 16, 16)])
        fp_dot = jnp.sum(acc)
        zv = stats_v[pl.ds(_N_EXP, 16)]
        inv_n = 1.0 / n_tokens
        auxst[...] = (jnp.full((16,), _AUX_COEF * _N_EXP * inv_n * inv_n
                               * fp_dot, jnp.float32)
                      + zv * (_Z_COEF * inv_n))
        pltpu.sync_copy(cfin, cnt_hbm)
        pltpu.sync_copy(auxst, aux_hbm)


def kernel(x, W):
    B, T, D = x.shape
    n = B * T
    x_flat = x.reshape(n, D)
    wt = W.T
    nb = n // _TB

    idx, prob, idx2, stats = pl.pallas_call(
        functools.partial(_router_body, tokens_per_tile=n // _SC_TILES),
        grid=(nb,),
        in_specs=[
            pl.BlockSpec((_TB, D), lambda i: (i, 0)),
            pl.BlockSpec((D, _N_EXP), lambda i: (0, 0)),
        ],
        out_specs=[
            pl.BlockSpec((_TB,), lambda i: (i,)),
            pl.BlockSpec((_TB,), lambda i: (i,)),
            pl.BlockSpec((_TB,), lambda i: (i,)),
            pl.BlockSpec((2 * _N_EXP,), lambda i: (0,)),
        ],
        out_shape=[
            jax.ShapeDtypeStruct((n,), jnp.int32),
            jax.ShapeDtypeStruct((n,), jnp.float32),
            jax.ShapeDtypeStruct((n,), jnp.int32),
            jax.ShapeDtypeStruct((2 * _N_EXP,), jnp.float32),
        ],
        scratch_shapes=[
            pltpu.VMEM((_N_EXP, _LANES), jnp.float32),
            pltpu.VMEM((1, _LANES), jnp.float32),
        ],
    )(x_flat, wt)

    rows_per_tile = n // (_LANES * _SC_TILES)
    sc_call = pl.kernel(
        functools.partial(_sc_body, rows_per_tile=rows_per_tile,
                          n_tokens=float(n)),
        out_type=[
            jax.ShapeDtypeStruct((_N_EXP,), jnp.int32),
            jax.ShapeDtypeStruct((16,), jnp.float32),
        ],
        mesh=plsc.VectorSubcoreMesh(core_axis_name="c", subcore_axis_name="s"),
        compiler_params=pltpu.CompilerParams(needs_layout_passes=False),
        scratch_types=[
            pltpu.VMEM((rows_per_tile, _LANES), jnp.int32),
            pltpu.VMEM((_LANES,), jnp.int32),
            pltpu.VMEM((2 * _N_EXP,), jnp.float32),
            pltpu.VMEM((_N_EXP,), jnp.int32),
            pltpu.VMEM((_SC_TILES * _N_EXP,), jnp.int32),
            pltpu.VMEM((16,), jnp.float32),
            pltpu.VMEM_SHARED((_SC_TILES * _N_EXP,), jnp.int32),
            pltpu.SemaphoreType.DMA,
        ],
    )
    counts, auxv = sc_call(idx2.reshape(n // _LANES, _LANES), stats)

    capacity = max(_MIN_CAP, math.ceil(_CAP_FACTOR * n / _N_EXP))
    return (idx, prob, counts, jnp.array(capacity, dtype=jnp.int32),
            auxv[0])


# speedup vs baseline: 1.0388x; 1.0311x over previous
import functools
import math

import jax
import jax.numpy as jnp
from jax.experimental import pallas as pl
from jax.experimental.pallas import tpu as pltpu
from jax.experimental.pallas import tpu_sc as plsc

_D_MODEL = 768
_N_EXP = 64
_Z_COEF = 0.001
_AUX_COEF = 0.01
_CAP_FACTOR = 1.0
_MIN_CAP = 4

_TB = 4096
_LANES = 128
_SC_TILES = 16


def _router_body(x_ref, wt_ref, idx_ref, prob_ref, idx2_ref, stats_ref,
                 p_acc, z_acc, *, tokens_per_tile):
    i = pl.program_id(0)
    nb = pl.num_programs(0)
    tb = x_ref.shape[0]

    lg = jax.lax.dot_general(
        wt_ref[...], x_ref[...],
        dimension_numbers=(((0,), (1,)), ((), ())),
        preferred_element_type=jnp.float32)
    m = jnp.max(lg, axis=0, keepdims=True)
    e = jnp.exp(lg - m)
    s = jnp.sum(e, axis=0, keepdims=True)
    eid = jax.lax.broadcasted_iota(jnp.int32, lg.shape, 0)
    amax = jnp.min(jnp.where(lg >= m, eid, _N_EXP), axis=0)
    idx_ref[...] = amax
    prob_ref[...] = 1.0 / s[0]
    lse = m[0] + jnp.log(s[0])

    gidx = i * tb + jax.lax.broadcasted_iota(jnp.int32, (tb,), 0)
    idx2_ref[...] = amax + (gidx // tokens_per_tile) * _N_EXP

    p_blk = jnp.sum((e * (1.0 / s)).reshape(_N_EXP, tb // _LANES, _LANES),
                    axis=1)
    z_blk = jnp.sum((lse * lse).reshape(tb // _LANES, _LANES),
                    axis=0, keepdims=True)

    @pl.when(i == 0)
    def _init():
        p_acc[...] = jnp.zeros_like(p_acc)
        z_acc[...] = jnp.zeros_like(z_acc)

    p_acc[...] += p_blk
    z_acc[...] += z_blk

    @pl.when(i == nb - 1)
    def _finish():
        p_vec = jnp.sum(p_acc[...], axis=1)
        z_sum = jnp.sum(z_acc[...])
        stats_ref[...] = jnp.concatenate(
            [p_vec, jnp.full((_N_EXP,), z_sum, jnp.float32)])


def _sc_body(idx_hbm, stats_hbm, cnt_hbm, aux_hbm,
             idx_v, ones_v, stats_v, cfin, hall_v, auxst, shared, sem,
             *, rows_per_tile, n_tokens):
    cid = jax.lax.axis_index("c")
    sid = jax.lax.axis_index("s")

    @pl.when(cid == 0)
    def _load():
        pltpu.sync_copy(idx_hbm.at[pl.ds(sid * rows_per_tile, rows_per_tile)],
                        idx_v)
        for t in range(_LANES // 16):
            ones_v[pl.ds(t * 16, 16)] = jnp.ones((16,), jnp.int32)
        for k in range(_N_EXP // 16):
            cfin[pl.ds(k * 16, 16)] = jnp.zeros((16,), jnp.int32)
        pltpu.sync_copy(cfin, shared.at[pl.ds(sid * _N_EXP, _N_EXP)])

    @pl.when((cid == 0) & (sid == 0))
    def _pref():
        pltpu.sync_copy(stats_hbm, stats_v)

    plsc.subcore_barrier()

    @pl.when(cid == 0)
    def _scatter():
        copies = [
            pltpu.make_async_copy(ones_v, shared.at[idx_v.at[j]], sem)
            for j in range(rows_per_tile)
        ]
        for c in copies:
            c.start(add=True)
        for c in copies:
            c.wait()

    plsc.subcore_barrier()

    @pl.when((cid == 0) & (sid == 0))
    def _final():
        pltpu.sync_copy(shared, hall_v)
        tot = []
        for k in range(_N_EXP // 16):
            acc_k = jnp.zeros((16,), jnp.int32)
            for r in range(_SC_TILES):
                acc_k = acc_k + hall_v[pl.ds(r * _N_EXP + k * 16, 16)]
            tot.append(acc_k)
        acc = jnp.zeros((16,), jnp.float32)
        for k in range(_N_EXP // 16):
            cfin[pl.ds(k * 16, 16)] = tot[k]
            acc = acc + (tot[k].astype(jnp.float32)
                         * stats_v[pl.ds(k * 16, 16)])
        fp_dot = jnp.sum(acc)
        zv = stats_v[pl.ds(_N_EXP, 16)]
        inv_n = 1.0 / n_tokens
        auxst[...] = (jnp.full((16,), _AUX_COEF * _N_EXP * inv_n * inv_n
                               * fp_dot, jnp.float32)
                      + zv * (_Z_COEF * inv_n))
        pltpu.sync_copy(cfin, cnt_hbm)
        pltpu.sync_copy(auxst, aux_hbm)


def kernel(x, W):
    B, T, D = x.shape
    n = B * T
    x_flat = x.reshape(n, D)
    wt = W.T
    nb = n // _TB

    idx, prob, idx2, stats = pl.pallas_call(
        functools.partial(_router_body, tokens_per_tile=n // _SC_TILES),
        grid=(nb,),
        in_specs=[
            pl.BlockSpec((_TB, D), lambda i: (i, 0)),
            pl.BlockSpec((D, _N_EXP), lambda i: (0, 0)),
        ],
        out_specs=[
            pl.BlockSpec((_TB,), lambda i: (i,)),
            pl.BlockSpec((_TB,), lambda i: (i,)),
            pl.BlockSpec((_TB,), lambda i: (i,)),
            pl.BlockSpec((2 * _N_EXP,), lambda i: (0,)),
        ],
        out_shape=[
            jax.ShapeDtypeStruct((n,), jnp.int32),
            jax.ShapeDtypeStruct((n,), jnp.float32),
            jax.ShapeDtypeStruct((n,), jnp.int32),
            jax.ShapeDtypeStruct((2 * _N_EXP,), jnp.float32),
        ],
        scratch_shapes=[
            pltpu.VMEM((_N_EXP, _LANES), jnp.float32),
            pltpu.VMEM((1, _LANES), jnp.float32),
        ],
    )(x_flat, wt)

    rows_per_tile = n // (_LANES * _SC_TILES)
    sc_call = pl.kernel(
        functools.partial(_sc_body, rows_per_tile=rows_per_tile,
                          n_tokens=float(n)),
        out_type=[
            jax.ShapeDtypeStruct((_N_EXP,), jnp.int32),
            jax.ShapeDtypeStruct((16,), jnp.float32),
        ],
        mesh=plsc.VectorSubcoreMesh(core_axis_name="c", subcore_axis_name="s",
                                    num_cores=1),
        compiler_params=pltpu.CompilerParams(needs_layout_passes=False),
        scratch_types=[
            pltpu.VMEM((rows_per_tile, _LANES), jnp.int32),
            pltpu.VMEM((_LANES,), jnp.int32),
            pltpu.VMEM((2 * _N_EXP,), jnp.float32),
            pltpu.VMEM((_N_EXP,), jnp.int32),
            pltpu.VMEM((_SC_TILES * _N_EXP,), jnp.int32),
            pltpu.VMEM((16,), jnp.float32),
            pltpu.VMEM_SHARED((_SC_TILES * _N_EXP,), jnp.int32),
            pltpu.SemaphoreType.DMA,
        ],
    )
    counts, auxv = sc_call(idx2.reshape(n // _LANES, _LANES), stats)

    capacity = max(_MIN_CAP, math.ceil(_CAP_FACTOR * n / _N_EXP))
    return (idx, prob, counts, jnp.array(capacity, dtype=jnp.int32),
            auxv[0])
